# Initial kernel scaffold; baseline (speedup 1.0000x reference)
import jax
import jax.numpy as jnp
from jax.experimental import pallas as pl


def kernel(box, cls):
    raise NotImplementedError("write your pallas kernel here")


# speedup vs baseline: 72.9248x; 72.9248x over previous
import dataclasses
import functools

import jax
import jax.numpy as jnp
from jax import lax
from jax.experimental import pallas as pl
from jax.experimental.pallas import tpu as pltpu
from jax.experimental.pallas import tpu_sc as plsc

_N = 5000
_C = 80
_B = 8
_PAD = 8
_NP = _N + _PAD
_NCH = _NP // 16
_MAXD = 300
_OUTP = 304
_NMS_TH = 0.5
_SCORE_TH = 0.05


def _score_label_body(cls_ref, s_ref, l_ref):
    c = cls_ref[0]
    m = jnp.max(c, axis=1)
    cls_ids = lax.broadcasted_iota(jnp.int32, (_N, _C), 1)
    lab = jnp.min(jnp.where(c == m[:, None], cls_ids, _C),
                  axis=1).astype(jnp.float32)
    s_ref[0, 0] = jnp.where(m > _SCORE_TH, m, -jnp.inf)
    l_ref[0, 0] = lab


def _scores_labels(cls):
    s3, l3 = pl.pallas_call(
        _score_label_body,
        grid=(_B,),
        in_specs=[pl.BlockSpec((1, _N, _C), lambda i: (i, 0, 0))],
        out_specs=[pl.BlockSpec((1, 1, _N), lambda i: (i, 0, 0)),
                   pl.BlockSpec((1, 1, _N), lambda i: (i, 0, 0))],
        out_shape=[jax.ShapeDtypeStruct((_B, 1, _N), jnp.float32),
                   jax.ShapeDtypeStruct((_B, 1, _N), jnp.float32)],
    )(cls)
    return s3[:, 0, :], l3[:, 0, :]


def _nms_body(x1h, y1h, x2h, y2h, sh, lh, oh,
              ox1h, oy1h, ox2h, oy2h, osh, olh,
              ux1, uy1, ux2, uy2, us, ul, ordv,
              sx1, sy1, sx2, sy2, ss, slb, sar, kpm,
              ox1, oy1, ox2, oy2, osv, olv):
    cid = lax.axis_index("c")
    sid = lax.axis_index("s")
    wid = sid * 2 + cid

    @pl.when(wid < _B)
    def _():
        img = wid
        pltpu.sync_copy(x1h.at[img], ux1)
        pltpu.sync_copy(y1h.at[img], uy1)
        pltpu.sync_copy(x2h.at[img], ux2)
        pltpu.sync_copy(y2h.at[img], uy2)
        pltpu.sync_copy(sh.at[img], us)
        pltpu.sync_copy(lh.at[img], ul)
        pltpu.sync_copy(oh.at[img], ordv)

        iota = lax.broadcasted_iota(jnp.int32, (16,), 0)

        @pl.loop(0, _NCH)
        def _(j):
            sl16 = pl.ds(j * 16, 16)
            idx = ordv[sl16]
            a = plsc.load_gather(ux1, [idx])
            b = plsc.load_gather(uy1, [idx])
            c = plsc.load_gather(ux2, [idx])
            d = plsc.load_gather(uy2, [idx])
            sv = plsc.load_gather(us, [idx])
            lv = plsc.load_gather(ul, [idx])
            sx1[sl16] = a
            sy1[sl16] = b
            sx2[sl16] = c
            sy2[sl16] = d
            ss[sl16] = sv
            slb[sl16] = lv
            sar[sl16] = (c - a) * (d - b)
            kpm[sl16] = (sv > -jnp.inf).astype(jnp.int32)

        @pl.loop(0, _OUTP // 16)
        def _(j):
            sl16 = pl.ds(j * 16, 16)
            neg = jnp.full((16,), -1.0, jnp.float32)
            ox1[sl16] = neg
            oy1[sl16] = neg
            ox2[sl16] = neg
            oy2[sl16] = neg
            osv[sl16] = neg
            olv[sl16] = neg

        def scan_cond(carry):
            pos, cnt, stop = carry
            return (pos < _N) & (cnt < _MAXD) & (stop == 0)

        def scan_body(carry):
            pos, cnt, stop = carry
            base = (pos // 16) * 16
            lane = pos - base
            lsel = iota == lane
            slb16 = pl.ds(base, 16)
            kvec = kpm[slb16]
            svec = ss[slb16]
            kp = jnp.max(jnp.where(lsel, kvec, 0))
            sp = jnp.max(jnp.where(lsel, svec, -jnp.inf))
            is_kept = kp > 0

            @pl.when(is_kept & (cnt < _MAXD - 1))
            def _():
                x1p = jnp.max(jnp.where(lsel, sx1[slb16], -jnp.inf))
                y1p = jnp.max(jnp.where(lsel, sy1[slb16], -jnp.inf))
                x2p = jnp.max(jnp.where(lsel, sx2[slb16], -jnp.inf))
                y2p = jnp.max(jnp.where(lsel, sy2[slb16], -jnp.inf))
                ap = jnp.max(jnp.where(lsel, sar[slb16], -jnp.inf))

                def chunk(j, carry_c):
                    slj = pl.ds(j * 16, 16)
                    gi = j * 16 + iota
                    xx1 = jnp.maximum(x1p, sx1[slj])
                    yy1 = jnp.maximum(y1p, sy1[slj])
                    xx2 = jnp.minimum(x2p, sx2[slj])
                    yy2 = jnp.minimum(y2p, sy2[slj])
                    inter = (jnp.maximum(xx2 - xx1, 0.0)
                             * jnp.maximum(yy2 - yy1, 0.0))
                    union = (ap + sar[slj]) - inter
                    iou = inter / jnp.maximum(union, 1e-12)
                    supp = (iou > _NMS_TH) & (gi > pos)
                    kpm[slj] = jnp.where(supp, 0, kpm[slj])
                    return carry_c

                lax.fori_loop(pos // 16, _NCH, chunk, jnp.int32(0))

            cnt2 = cnt + jnp.where(is_kept, 1, 0).astype(jnp.int32)
            stop2 = jnp.where(sp == -jnp.inf, 1, 0).astype(jnp.int32)
            return (pos + 1, cnt2, stop2)

        lax.while_loop(scan_cond, scan_body,
                       (jnp.int32(0), jnp.int32(0), jnp.int32(0)))

        def comp(j, off):
            slj = pl.ds(j * 16, 16)
            m = kpm[slj]
            csum = plsc.cumsum(m)
            dest = off + csum - 1
            sel = (m > 0) & (dest < _MAXD)
            plsc.store_scatter(ox1, [dest], sx1[slj], mask=sel)
            plsc.store_scatter(oy1, [dest], sy1[slj], mask=sel)
            plsc.store_scatter(ox2, [dest], sx2[slj], mask=sel)
            plsc.store_scatter(oy2, [dest], sy2[slj], mask=sel)
            plsc.store_scatter(osv, [dest], ss[slj], mask=sel)
            plsc.store_scatter(olv, [dest], slb[slj], mask=sel)
            return off + jnp.max(csum)

        lax.fori_loop(0, _NCH, comp, jnp.int32(0))

        pltpu.sync_copy(ox1, ox1h.at[img])
        pltpu.sync_copy(oy1, oy1h.at[img])
        pltpu.sync_copy(ox2, ox2h.at[img])
        pltpu.sync_copy(oy2, oy2h.at[img])
        pltpu.sync_copy(osv, osh.at[img])
        pltpu.sync_copy(olv, olh.at[img])


_sc_params = pltpu.CompilerParams()
if "needs_layout_passes" in pltpu.CompilerParams.__dataclass_fields__:
    _sc_params = dataclasses.replace(_sc_params, needs_layout_passes=False)

_nms_call = functools.partial(
    pl.kernel,
    compiler_params=_sc_params,
    out_type=[jax.ShapeDtypeStruct((_B, _OUTP), jnp.float32)
              for _ in range(6)],
    mesh=plsc.VectorSubcoreMesh(core_axis_name="c", subcore_axis_name="s"),
    scratch_types=(
        [pltpu.VMEM((_NP,), jnp.float32) for _ in range(6)]
        + [pltpu.VMEM((_NP,), jnp.int32)]
        + [pltpu.VMEM((_NP,), jnp.float32) for _ in range(7)]
        + [pltpu.VMEM((_NP,), jnp.int32)]
        + [pltpu.VMEM((_OUTP,), jnp.float32) for _ in range(6)]
    ),
)(_nms_body)


def kernel(box, cls):
    s, lab = _scores_labels(cls)
    order = jnp.argsort(-s, axis=1).astype(jnp.int32)
    pad_idx = jnp.broadcast_to(jnp.arange(_N, _NP, dtype=jnp.int32),
                               (_B, _PAD))
    order_p = jnp.concatenate([order, pad_idx], axis=1)
    s_p = jnp.concatenate(
        [s, jnp.full((_B, _PAD), -jnp.inf, jnp.float32)], axis=1)
    lab_p = jnp.concatenate([lab, jnp.zeros((_B, _PAD), jnp.float32)], axis=1)
    box_p = jnp.concatenate(
        [box, jnp.zeros((_B, _PAD, 4), jnp.float32)], axis=1)
    x1 = box_p[:, :, 0]
    y1 = box_p[:, :, 1]
    x2 = box_p[:, :, 2]
    y2 = box_p[:, :, 3]
    ox1, oy1, ox2, oy2, osv, olv = _nms_call(
        x1, y1, x2, y2, s_p, lab_p, order_p)
    boxes = jnp.stack([ox1[:, :_MAXD], oy1[:, :_MAXD],
                       ox2[:, :_MAXD], oy2[:, :_MAXD]], axis=-1)
    scores = osv[:, :_MAXD, None]
    labels = olv[:, :_MAXD, None]
    return boxes, scores, labels


# speedup vs baseline: 73.2040x; 1.0038x over previous
import dataclasses
import functools

import jax
import jax.numpy as jnp
from jax import lax
from jax.experimental import pallas as pl
from jax.experimental.pallas import tpu as pltpu
from jax.experimental.pallas import tpu_sc as plsc

_N = 5000
_C = 80
_B = 8
_PAD = 56
_NP = _N + _PAD
_NCH = _NP // 16
_NB64 = _NP // 64
_MAXD = 300
_OUTP = 304
_NMS_TH = 0.5
_SCORE_TH = 0.05
_NEG_INF = float("-inf")


def _score_label_body(cls_ref, s_ref, l_ref):
    c = cls_ref[0]
    m = jnp.max(c, axis=1)
    cls_ids = lax.broadcasted_iota(jnp.int32, (_N, _C), 1)
    lab = jnp.min(jnp.where(c == m[:, None], cls_ids, _C),
                  axis=1).astype(jnp.float32)
    s_ref[0, 0] = jnp.where(m > _SCORE_TH, m, -jnp.inf)
    l_ref[0, 0] = lab


def _scores_labels(cls):
    s3, l3 = pl.pallas_call(
        _score_label_body,
        grid=(_B,),
        in_specs=[pl.BlockSpec((1, _N, _C), lambda i: (i, 0, 0))],
        out_specs=[pl.BlockSpec((1, 1, _N), lambda i: (i, 0, 0)),
                   pl.BlockSpec((1, 1, _N), lambda i: (i, 0, 0))],
        out_shape=[jax.ShapeDtypeStruct((_B, 1, _N), jnp.float32),
                   jax.ShapeDtypeStruct((_B, 1, _N), jnp.float32)],
    )(cls)
    return s3[:, 0, :], l3[:, 0, :]


def _nms_body(x1h, y1h, x2h, y2h, sh, lh, oh,
              ox1h, oy1h, ox2h, oy2h, osh, olh,
              ux1, uy1, ux2, uy2, us, ul, ordv,
              sx1, sy1, sx2, sy2, ss, slb,
              ox1, oy1, ox2, oy2, osv, olv):
    cid = lax.axis_index("c")
    sid = lax.axis_index("s")
    wid = sid * 2 + cid

    @pl.when(wid < _B)
    def _():
        img = wid
        pltpu.sync_copy(x1h.at[img], ux1)
        pltpu.sync_copy(y1h.at[img], uy1)
        pltpu.sync_copy(x2h.at[img], ux2)
        pltpu.sync_copy(y2h.at[img], uy2)
        pltpu.sync_copy(sh.at[img], us)
        pltpu.sync_copy(lh.at[img], ul)
        pltpu.sync_copy(oh.at[img], ordv)

        iota = lax.broadcasted_iota(jnp.int32, (16,), 0)

        def init(j, nvalid):
            sl16 = pl.ds(j * 16, 16)
            idx = ordv[sl16]
            sx1[sl16] = plsc.load_gather(ux1, [idx])
            sy1[sl16] = plsc.load_gather(uy1, [idx])
            sx2[sl16] = plsc.load_gather(ux2, [idx])
            sy2[sl16] = plsc.load_gather(uy2, [idx])
            sv = plsc.load_gather(us, [idx])
            ss[sl16] = sv
            slb[sl16] = plsc.load_gather(ul, [idx])
            return nvalid + jnp.sum((sv > _NEG_INF).astype(jnp.int32))

        n_valid = lax.fori_loop(0, _NCH, init, jnp.int32(0))

        @pl.loop(0, _OUTP // 16)
        def _(j):
            sl16 = pl.ds(j * 16, 16)
            neg = jnp.full((16,), -1.0, jnp.float32)
            ox1[sl16] = neg
            oy1[sl16] = neg
            ox2[sl16] = neg
            oy2[sl16] = neg
            osv[sl16] = neg
            olv[sl16] = neg

        def scan_cond(carry):
            pos, cnt = carry
            return (pos < n_valid) & (cnt < _MAXD)

        def scan_body(carry):
            pos, cnt = carry
            base = (pos // 16) * 16
            lane = pos - base
            lsel = iota == lane
            sp = jnp.max(jnp.where(lsel, ss[pl.ds(base, 16)], -jnp.inf))
            is_kept = sp > _NEG_INF

            @pl.when(is_kept & (cnt < _MAXD - 1))
            def _():
                slb16 = pl.ds(base, 16)
                x1p = jnp.max(jnp.where(lsel, sx1[slb16], -jnp.inf))
                y1p = jnp.max(jnp.where(lsel, sy1[slb16], -jnp.inf))
                x2p = jnp.max(jnp.where(lsel, sx2[slb16], -jnp.inf))
                y2p = jnp.max(jnp.where(lsel, sy2[slb16], -jnp.inf))
                ap = (x2p - x1p) * (y2p - y1p)

                def sub_chunk(start, masked):
                    slj = pl.ds(start, 16)
                    x1c = sx1[slj]
                    y1c = sy1[slj]
                    x2c = sx2[slj]
                    y2c = sy2[slj]
                    sc = ss[slj]
                    xx1 = jnp.maximum(x1p, x1c)
                    yy1 = jnp.maximum(y1p, y1c)
                    xx2 = jnp.minimum(x2p, x2c)
                    yy2 = jnp.minimum(y2p, y2c)
                    inter = (jnp.maximum(xx2 - xx1, 0.0)
                             * jnp.maximum(yy2 - yy1, 0.0))
                    union = (ap + (x2c - x1c) * (y2c - y1c)) - inter
                    iou = inter / jnp.maximum(union, 1e-12)
                    supp = iou > _NMS_TH
                    if masked:
                        supp = supp & ((start + iota) > pos)
                    ss[slj] = jnp.where(supp, -jnp.inf, sc)

                blk0 = (pos // 64) * 64
                for k in range(4):
                    sub_chunk(blk0 + k * 16, True)

                def block(j, c):
                    for k in range(4):
                        sub_chunk(j * 64 + k * 16, False)
                    return c

                lax.fori_loop(pos // 64 + 1, _NB64, block, jnp.int32(0))

            cnt2 = cnt + jnp.where(is_kept, 1, 0).astype(jnp.int32)
            return (pos + 1, cnt2)

        lax.while_loop(scan_cond, scan_body, (jnp.int32(0), jnp.int32(0)))

        def comp(j, off):
            slj = pl.ds(j * 16, 16)
            m = (ss[slj] > _NEG_INF).astype(jnp.int32)
            csum = plsc.cumsum(m)
            dest = off + csum - 1
            sel = (m > 0) & (dest < _MAXD)
            plsc.store_scatter(ox1, [dest], sx1[slj], mask=sel)
            plsc.store_scatter(oy1, [dest], sy1[slj], mask=sel)
            plsc.store_scatter(ox2, [dest], sx2[slj], mask=sel)
            plsc.store_scatter(oy2, [dest], sy2[slj], mask=sel)
            plsc.store_scatter(osv, [dest], ss[slj], mask=sel)
            plsc.store_scatter(olv, [dest], slb[slj], mask=sel)
            return off + jnp.max(csum)

        lax.fori_loop(0, _NCH, comp, jnp.int32(0))

        pltpu.sync_copy(ox1, ox1h.at[img])
        pltpu.sync_copy(oy1, oy1h.at[img])
        pltpu.sync_copy(ox2, ox2h.at[img])
        pltpu.sync_copy(oy2, oy2h.at[img])
        pltpu.sync_copy(osv, osh.at[img])
        pltpu.sync_copy(olv, olh.at[img])


_sc_params = pltpu.CompilerParams()
if "needs_layout_passes" in pltpu.CompilerParams.__dataclass_fields__:
    _sc_params = dataclasses.replace(_sc_params, needs_layout_passes=False)

_nms_call = functools.partial(
    pl.kernel,
    compiler_params=_sc_params,
    out_type=[jax.ShapeDtypeStruct((_B, _OUTP), jnp.float32)
              for _ in range(6)],
    mesh=plsc.VectorSubcoreMesh(core_axis_name="c", subcore_axis_name="s"),
    scratch_types=(
        [pltpu.VMEM((_NP,), jnp.float32) for _ in range(6)]
        + [pltpu.VMEM((_NP,), jnp.int32)]
        + [pltpu.VMEM((_NP,), jnp.float32) for _ in range(6)]
        + [pltpu.VMEM((_OUTP,), jnp.float32) for _ in range(6)]
    ),
)(_nms_body)


def kernel(box, cls):
    s, lab = _scores_labels(cls)
    order = jnp.argsort(-s, axis=1).astype(jnp.int32)
    pad_idx = jnp.broadcast_to(jnp.arange(_N, _NP, dtype=jnp.int32),
                               (_B, _PAD))
    order_p = jnp.concatenate([order, pad_idx], axis=1)
    s_p = jnp.concatenate(
        [s, jnp.full((_B, _PAD), -jnp.inf, jnp.float32)], axis=1)
    lab_p = jnp.concatenate([lab, jnp.zeros((_B, _PAD), jnp.float32)], axis=1)
    box_p = jnp.concatenate(
        [box, jnp.zeros((_B, _PAD, 4), jnp.float32)], axis=1)
    x1 = box_p[:, :, 0]
    y1 = box_p[:, :, 1]
    x2 = box_p[:, :, 2]
    y2 = box_p[:, :, 3]
    ox1, oy1, ox2, oy2, osv, olv = _nms_call(
        x1, y1, x2, y2, s_p, lab_p, order_p)
    boxes = jnp.stack([ox1[:, :_MAXD], oy1[:, :_MAXD],
                       ox2[:, :_MAXD], oy2[:, :_MAXD]], axis=-1)
    scores = osv[:, :_MAXD, None]
    labels = olv[:, :_MAXD, None]
    return boxes, scores, labels


# speedup vs baseline: 184.9687x; 2.5268x over previous
import dataclasses
import functools

import jax
import jax.numpy as jnp
from jax import lax
from jax.experimental import pallas as pl
from jax.experimental.pallas import tpu as pltpu
from jax.experimental.pallas import tpu_sc as plsc

_N = 5000
_C = 80
_B = 8
_PAD = 56
_NP = _N + _PAD
_NCH = _NP // 16
_NB64 = _NP // 64
_MAXD = 300
_OUTP = 304
_NMS_TH = 0.5
_SCORE_TH = 0.05
_NEG_INF = float("-inf")


def _score_label_body(cls_ref, s_ref, l_ref):
    c = cls_ref[0]
    m = jnp.max(c, axis=1)
    cls_ids = lax.broadcasted_iota(jnp.int32, (_N, _C), 1)
    lab = jnp.min(jnp.where(c == m[:, None], cls_ids, _C),
                  axis=1).astype(jnp.float32)
    s_ref[0, 0] = jnp.where(m > _SCORE_TH, m, -jnp.inf)
    l_ref[0, 0] = lab


def _scores_labels(cls):
    s3, l3 = pl.pallas_call(
        _score_label_body,
        grid=(_B,),
        in_specs=[pl.BlockSpec((1, _N, _C), lambda i: (i, 0, 0))],
        out_specs=[pl.BlockSpec((1, 1, _N), lambda i: (i, 0, 0)),
                   pl.BlockSpec((1, 1, _N), lambda i: (i, 0, 0))],
        out_shape=[jax.ShapeDtypeStruct((_B, 1, _N), jnp.float32),
                   jax.ShapeDtypeStruct((_B, 1, _N), jnp.float32)],
    )(cls)
    return s3[:, 0, :], l3[:, 0, :]


def _nms_body(x1h, y1h, x2h, y2h, sh, lh, oh,
              ox1h, oy1h, ox2h, oy2h, osh, olh,
              ux1, uy1, ux2, uy2, us, ul, ordv,
              sx1, sy1, sx2, sy2, ss, slb,
              ox1, oy1, ox2, oy2, osv, olv):
    cid = lax.axis_index("c")
    sid = lax.axis_index("s")
    wid = sid * 2 + cid

    @pl.when(wid < _B)
    def _():
        img = wid
        pltpu.sync_copy(x1h.at[img], ux1)
        pltpu.sync_copy(y1h.at[img], uy1)
        pltpu.sync_copy(x2h.at[img], ux2)
        pltpu.sync_copy(y2h.at[img], uy2)
        pltpu.sync_copy(sh.at[img], us)
        pltpu.sync_copy(lh.at[img], ul)
        pltpu.sync_copy(oh.at[img], ordv)

        iota = lax.broadcasted_iota(jnp.int32, (16,), 0)

        def init(j, nvalid):
            sl16 = pl.ds(j * 16, 16)
            idx = ordv[sl16]
            sx1[sl16] = plsc.load_gather(ux1, [idx])
            sy1[sl16] = plsc.load_gather(uy1, [idx])
            sx2[sl16] = plsc.load_gather(ux2, [idx])
            sy2[sl16] = plsc.load_gather(uy2, [idx])
            sv = plsc.load_gather(us, [idx])
            ss[sl16] = sv
            slb[sl16] = plsc.load_gather(ul, [idx])
            return nvalid + jnp.sum((sv > _NEG_INF).astype(jnp.int32))

        n_valid = lax.fori_loop(0, _NCH, init, jnp.int32(0))

        @pl.loop(0, _OUTP // 16)
        def _(j):
            sl16 = pl.ds(j * 16, 16)
            neg = jnp.full((16,), -1.0, jnp.float32)
            ox1[sl16] = neg
            oy1[sl16] = neg
            ox2[sl16] = neg
            oy2[sl16] = neg
            osv[sl16] = neg
            olv[sl16] = neg

        def scan_cond(carry):
            pos, cnt = carry
            return (pos < n_valid) & (cnt < _MAXD)

        def scan_body(carry):
            pos, cnt = carry
            base = (pos // 16) * 16
            lane = pos - base
            lsel = iota == lane
            sp = jnp.max(jnp.where(lsel, ss[pl.ds(base, 16)], -jnp.inf))
            is_kept = sp > _NEG_INF

            @pl.when(is_kept & (cnt < _MAXD - 1))
            def _():
                slb16 = pl.ds(base, 16)
                x1p = jnp.max(jnp.where(lsel, sx1[slb16], -jnp.inf))
                y1p = jnp.max(jnp.where(lsel, sy1[slb16], -jnp.inf))
                x2p = jnp.max(jnp.where(lsel, sx2[slb16], -jnp.inf))
                y2p = jnp.max(jnp.where(lsel, sy2[slb16], -jnp.inf))
                ap = (x2p - x1p) * (y2p - y1p)
                ninf = jnp.full((16,), -jnp.inf, jnp.float32)

                def block64(jbase, masked):
                    xs = []
                    for k in range(4):
                        slj = pl.ds(jbase + k * 16, 16)
                        xs.append((sx1[slj], sy1[slj], sx2[slj], sy2[slj]))
                    supps = []
                    for k in range(4):
                        x1c, y1c, x2c, y2c = xs[k]
                        xx1 = jnp.maximum(x1p, x1c)
                        yy1 = jnp.maximum(y1p, y1c)
                        xx2 = jnp.minimum(x2p, x2c)
                        yy2 = jnp.minimum(y2p, y2c)
                        inter = (jnp.maximum(xx2 - xx1, 0.0)
                                 * jnp.maximum(yy2 - yy1, 0.0))
                        union = (ap + (x2c - x1c) * (y2c - y1c)) - inter
                        supp = inter > _NMS_TH * jnp.maximum(union, 1e-12)
                        if masked:
                            supp = supp & ((jbase + k * 16 + iota) > pos)
                        supps.append(supp)
                    for k in range(4):
                        idx = jbase + k * 16 + iota
                        plsc.store_scatter(ss, [idx], ninf, mask=supps[k])

                block64((pos // 64) * 64, True)

                def block(j, c):
                    block64(j * 64, False)
                    return c

                lax.fori_loop(pos // 64 + 1, _NB64, block, jnp.int32(0))

            cnt2 = cnt + jnp.where(is_kept, 1, 0).astype(jnp.int32)
            return (pos + 1, cnt2)

        lax.while_loop(scan_cond, scan_body, (jnp.int32(0), jnp.int32(0)))

        def comp(j, off):
            slj = pl.ds(j * 16, 16)
            m = (ss[slj] > _NEG_INF).astype(jnp.int32)
            csum = plsc.cumsum(m)
            dest = off + csum - 1
            sel = (m > 0) & (dest < _MAXD)
            plsc.store_scatter(ox1, [dest], sx1[slj], mask=sel)
            plsc.store_scatter(oy1, [dest], sy1[slj], mask=sel)
            plsc.store_scatter(ox2, [dest], sx2[slj], mask=sel)
            plsc.store_scatter(oy2, [dest], sy2[slj], mask=sel)
            plsc.store_scatter(osv, [dest], ss[slj], mask=sel)
            plsc.store_scatter(olv, [dest], slb[slj], mask=sel)
            return off + jnp.max(csum)

        lax.fori_loop(0, _NCH, comp, jnp.int32(0))

        pltpu.sync_copy(ox1, ox1h.at[img])
        pltpu.sync_copy(oy1, oy1h.at[img])
        pltpu.sync_copy(ox2, ox2h.at[img])
        pltpu.sync_copy(oy2, oy2h.at[img])
        pltpu.sync_copy(osv, osh.at[img])
        pltpu.sync_copy(olv, olh.at[img])


_sc_params = pltpu.CompilerParams()
if "needs_layout_passes" in pltpu.CompilerParams.__dataclass_fields__:
    _sc_params = dataclasses.replace(_sc_params, needs_layout_passes=False)

_nms_call = functools.partial(
    pl.kernel,
    compiler_params=_sc_params,
    out_type=[jax.ShapeDtypeStruct((_B, _OUTP), jnp.float32)
              for _ in range(6)],
    mesh=plsc.VectorSubcoreMesh(core_axis_name="c", subcore_axis_name="s"),
    scratch_types=(
        [pltpu.VMEM((_NP,), jnp.float32) for _ in range(6)]
        + [pltpu.VMEM((_NP,), jnp.int32)]
        + [pltpu.VMEM((_NP,), jnp.float32) for _ in range(6)]
        + [pltpu.VMEM((_OUTP,), jnp.float32) for _ in range(6)]
    ),
)(_nms_body)


def kernel(box, cls):
    s, lab = _scores_labels(cls)
    order = jnp.argsort(-s, axis=1).astype(jnp.int32)
    pad_idx = jnp.broadcast_to(jnp.arange(_N, _NP, dtype=jnp.int32),
                               (_B, _PAD))
    order_p = jnp.concatenate([order, pad_idx], axis=1)
    s_p = jnp.concatenate(
        [s, jnp.full((_B, _PAD), -jnp.inf, jnp.float32)], axis=1)
    lab_p = jnp.concatenate([lab, jnp.zeros((_B, _PAD), jnp.float32)], axis=1)
    box_p = jnp.concatenate(
        [box, jnp.zeros((_B, _PAD, 4), jnp.float32)], axis=1)
    x1 = box_p[:, :, 0]
    y1 = box_p[:, :, 1]
    x2 = box_p[:, :, 2]
    y2 = box_p[:, :, 3]
    ox1, oy1, ox2, oy2, osv, olv = _nms_call(
        x1, y1, x2, y2, s_p, lab_p, order_p)
    boxes = jnp.stack([ox1[:, :_MAXD], oy1[:, :_MAXD],
                       ox2[:, :_MAXD], oy2[:, :_MAXD]], axis=-1)
    scores = osv[:, :_MAXD, None]
    labels = olv[:, :_MAXD, None]
    return boxes, scores, labels


# speedup vs baseline: 389.3770x; 2.1051x over previous
import dataclasses
import functools

import jax
import jax.numpy as jnp
from jax import lax
from jax.experimental import pallas as pl
from jax.experimental.pallas import tpu as pltpu
from jax.experimental.pallas import tpu_sc as plsc

_N = 5000
_C = 80
_B = 8
_PAD = 56
_NP = _N + _PAD
_NCH = _NP // 16
_NB64 = _NP // 64
_R = 512
_RB64 = _R // 64
_STRIPE = -(-(_NB64 - _RB64) // 4)
_MAXD = 300
_OUTP = 304
_NMS_TH = 0.5
_SCORE_TH = 0.05
_NEG_INF = float("-inf")


def _score_label_body(cls_ref, s_ref, l_ref):
    c = cls_ref[0]
    m = jnp.max(c, axis=1)
    cls_ids = lax.broadcasted_iota(jnp.int32, (_N, _C), 1)
    lab = jnp.min(jnp.where(c == m[:, None], cls_ids, _C),
                  axis=1).astype(jnp.float32)
    s_ref[0, 0] = jnp.where(m > _SCORE_TH, m, -jnp.inf)
    l_ref[0, 0] = lab


def _scores_labels(cls):
    s3, l3 = pl.pallas_call(
        _score_label_body,
        grid=(_B,),
        in_specs=[pl.BlockSpec((1, _N, _C), lambda i: (i, 0, 0))],
        out_specs=[pl.BlockSpec((1, 1, _N), lambda i: (i, 0, 0)),
                   pl.BlockSpec((1, 1, _N), lambda i: (i, 0, 0))],
        out_shape=[jax.ShapeDtypeStruct((_B, 1, _N), jnp.float32),
                   jax.ShapeDtypeStruct((_B, 1, _N), jnp.float32)],
    )(cls)
    return s3[:, 0, :], l3[:, 0, :]


def _nms_body(x1h, y1h, x2h, y2h, sh, lh, oh,
              ox1h, oy1h, ox2h, oy2h, osh, olh,
              ux1, uy1, ux2, uy2, us, ul, ordv,
              sx1, sy1, sx2, sy2, ss, slb,
              ox1, oy1, ox2, oy2, osv, olv):
    cid = lax.axis_index("c")
    sid = lax.axis_index("s")
    wid = sid * 2 + cid
    img = wid // 4
    sub = wid % 4
    is_leader = sub == 0
    my_lo = _RB64 + sub * _STRIPE
    my_hi = jnp.minimum(my_lo + _STRIPE, _NB64)

    pltpu.sync_copy(x1h.at[img], ux1)
    pltpu.sync_copy(y1h.at[img], uy1)
    pltpu.sync_copy(x2h.at[img], ux2)
    pltpu.sync_copy(y2h.at[img], uy2)
    pltpu.sync_copy(sh.at[img], us)
    pltpu.sync_copy(lh.at[img], ul)
    pltpu.sync_copy(oh.at[img], ordv)

    iota = lax.broadcasted_iota(jnp.int32, (16,), 0)
    ninf = jnp.full((16,), -jnp.inf, jnp.float32)

    def init(j, nvalid):
        sl16 = pl.ds(j * 16, 16)
        idx = ordv[sl16]
        sx1[sl16] = plsc.load_gather(ux1, [idx])
        sy1[sl16] = plsc.load_gather(uy1, [idx])
        sx2[sl16] = plsc.load_gather(ux2, [idx])
        sy2[sl16] = plsc.load_gather(uy2, [idx])
        sv = plsc.load_gather(us, [idx])
        ss[sl16] = sv
        slb[sl16] = plsc.load_gather(ul, [idx])
        return nvalid + jnp.sum((sv > _NEG_INF).astype(jnp.int32))

    n_valid = lax.fori_loop(0, _NCH, init, jnp.int32(0))

    @pl.when(is_leader)
    def _():
        @pl.loop(0, _OUTP // 16)
        def _(j):
            sl16 = pl.ds(j * 16, 16)
            neg = jnp.full((16,), -1.0, jnp.float32)
            ox1[sl16] = neg
            oy1[sl16] = neg
            ox2[sl16] = neg
            oy2[sl16] = neg
            osv[sl16] = neg
            olv[sl16] = neg

    def extract(pos):
        base = (pos // 16) * 16
        lsel = iota == (pos - base)
        sl16 = pl.ds(base, 16)
        sp = jnp.max(jnp.where(lsel, ss[sl16], -jnp.inf))
        x1p = jnp.max(jnp.where(lsel, sx1[sl16], -jnp.inf))
        y1p = jnp.max(jnp.where(lsel, sy1[sl16], -jnp.inf))
        x2p = jnp.max(jnp.where(lsel, sx2[sl16], -jnp.inf))
        y2p = jnp.max(jnp.where(lsel, sy2[sl16], -jnp.inf))
        return sp, (x1p, y1p, x2p, y2p, (x2p - x1p) * (y2p - y1p))

    def block64(box_p, jbase, masked, pos):
        x1p, y1p, x2p, y2p, ap = box_p
        xs = []
        for k in range(4):
            slj = pl.ds(jbase + k * 16, 16)
            xs.append((sx1[slj], sy1[slj], sx2[slj], sy2[slj]))
        supps = []
        for k in range(4):
            x1c, y1c, x2c, y2c = xs[k]
            xx1 = jnp.maximum(x1p, x1c)
            yy1 = jnp.maximum(y1p, y1c)
            xx2 = jnp.minimum(x2p, x2c)
            yy2 = jnp.minimum(y2p, y2c)
            inter = (jnp.maximum(xx2 - xx1, 0.0)
                     * jnp.maximum(yy2 - yy1, 0.0))
            union = (ap + (x2c - x1c) * (y2c - y1c)) - inter
            supp = inter > _NMS_TH * jnp.maximum(union, 1e-12)
            if masked:
                supp = supp & ((jbase + k * 16 + iota) > pos)
            supps.append(supp)
        for k in range(4):
            idx = jbase + k * 16 + iota
            plsc.store_scatter(ss, [idx], ninf, mask=supps[k])

    def make_scan(ranges_fn):
        def scan_body(carry):
            pos, cnt = carry
            sp, box_p = extract(pos)
            is_kept = sp > _NEG_INF

            @pl.when(is_kept & (cnt < _MAXD - 1))
            def _():
                block64(box_p, (pos // 64) * 64, True, pos)

                def blk(j, c):
                    block64(box_p, j * 64, False, pos)
                    return c

                for lo, hi in ranges_fn(pos):
                    lax.fori_loop(lo, hi, blk, jnp.int32(0))

            cnt2 = cnt + jnp.where(is_kept, 1, 0).astype(jnp.int32)
            return (pos + 1, cnt2)
        return scan_body

    scan_a = make_scan(lambda pos: [(pos // 64 + 1, _RB64),
                                    (my_lo, my_hi)])
    lim_a = jnp.minimum(n_valid, _R)
    pos_a, cnt_a = lax.while_loop(
        lambda c: (c[0] < lim_a) & (c[1] < _MAXD),
        scan_a, (jnp.int32(0), jnp.int32(0)))

    @pl.when(is_leader & (cnt_a < _MAXD) & (n_valid > _R))
    def _():
        def resup(p2, c):
            sp, box_p = extract(p2)

            @pl.when(sp > _NEG_INF)
            def _():
                def blk(j, cc):
                    block64(box_p, j * 64, False, p2)
                    return cc
                lax.fori_loop(_RB64, _NB64, blk, jnp.int32(0))
            return c

        lax.fori_loop(0, _R, resup, jnp.int32(0))

        scan_b = make_scan(lambda pos: [(pos // 64 + 1, _NB64)])
        lax.while_loop(
            lambda c: (c[0] < n_valid) & (c[1] < _MAXD),
            scan_b, (jnp.int32(_R), cnt_a))

    @pl.when(is_leader)
    def _():
        def comp(j, off):
            slj = pl.ds(j * 16, 16)
            m = (ss[slj] > _NEG_INF).astype(jnp.int32)
            csum = plsc.cumsum(m)
            dest = off + csum - 1
            sel = (m > 0) & (dest < _MAXD)
            plsc.store_scatter(ox1, [dest], sx1[slj], mask=sel)
            plsc.store_scatter(oy1, [dest], sy1[slj], mask=sel)
            plsc.store_scatter(ox2, [dest], sx2[slj], mask=sel)
            plsc.store_scatter(oy2, [dest], sy2[slj], mask=sel)
            plsc.store_scatter(osv, [dest], ss[slj], mask=sel)
            plsc.store_scatter(olv, [dest], slb[slj], mask=sel)
            return off + jnp.max(csum)

        lax.fori_loop(0, _NCH, comp, jnp.int32(0))

        pltpu.sync_copy(ox1, ox1h.at[img])
        pltpu.sync_copy(oy1, oy1h.at[img])
        pltpu.sync_copy(ox2, ox2h.at[img])
        pltpu.sync_copy(oy2, oy2h.at[img])
        pltpu.sync_copy(osv, osh.at[img])
        pltpu.sync_copy(olv, olh.at[img])


_sc_params = pltpu.CompilerParams()
if "needs_layout_passes" in pltpu.CompilerParams.__dataclass_fields__:
    _sc_params = dataclasses.replace(_sc_params, needs_layout_passes=False)

_nms_call = functools.partial(
    pl.kernel,
    compiler_params=_sc_params,
    out_type=[jax.ShapeDtypeStruct((_B, _OUTP), jnp.float32)
              for _ in range(6)],
    mesh=plsc.VectorSubcoreMesh(core_axis_name="c", subcore_axis_name="s"),
    scratch_types=(
        [pltpu.VMEM((_NP,), jnp.float32) for _ in range(6)]
        + [pltpu.VMEM((_NP,), jnp.int32)]
        + [pltpu.VMEM((_NP,), jnp.float32) for _ in range(6)]
        + [pltpu.VMEM((_OUTP,), jnp.float32) for _ in range(6)]
    ),
)(_nms_body)


def kernel(box, cls):
    s, lab = _scores_labels(cls)
    order = jnp.argsort(-s, axis=1).astype(jnp.int32)
    pad_idx = jnp.broadcast_to(jnp.arange(_N, _NP, dtype=jnp.int32),
                               (_B, _PAD))
    order_p = jnp.concatenate([order, pad_idx], axis=1)
    s_p = jnp.concatenate(
        [s, jnp.full((_B, _PAD), -jnp.inf, jnp.float32)], axis=1)
    lab_p = jnp.concatenate([lab, jnp.zeros((_B, _PAD), jnp.float32)], axis=1)
    box_p = jnp.concatenate(
        [box, jnp.zeros((_B, _PAD, 4), jnp.float32)], axis=1)
    x1 = box_p[:, :, 0]
    y1 = box_p[:, :, 1]
    x2 = box_p[:, :, 2]
    y2 = box_p[:, :, 3]
    ox1, oy1, ox2, oy2, osv, olv = _nms_call(
        x1, y1, x2, y2, s_p, lab_p, order_p)
    boxes = jnp.stack([ox1[:, :_MAXD], oy1[:, :_MAXD],
                       ox2[:, :_MAXD], oy2[:, :_MAXD]], axis=-1)
    scores = osv[:, :_MAXD, None]
    labels = olv[:, :_MAXD, None]
    return boxes, scores, labels


# speedup vs baseline: 424.2759x; 1.0896x over previous
import dataclasses
import functools

import jax
import jax.numpy as jnp
from jax import lax
from jax.experimental import pallas as pl
from jax.experimental.pallas import tpu as pltpu
from jax.experimental.pallas import tpu_sc as plsc

_N = 5000
_C = 80
_B = 8
_PAD = 120
_NP = _N + _PAD
_NCH = _NP // 16
_BLK = 128
_NB = _NP // _BLK
_R = 512
_RB = _R // _BLK
_STRIPE = (_NB - _RB) // 4
_MAXD = 300
_OUTP = 304
_NMS_TH = 0.5
_SCORE_TH = 0.05
_NEG_INF = float("-inf")


def _score_label_body(cls_ref, s_ref, l_ref):
    c = cls_ref[0]
    m = jnp.max(c, axis=1)
    cls_ids = lax.broadcasted_iota(jnp.int32, (_N, _C), 1)
    lab = jnp.min(jnp.where(c == m[:, None], cls_ids, _C),
                  axis=1).astype(jnp.float32)
    s_ref[0, 0] = jnp.where(m > _SCORE_TH, m, -jnp.inf)
    l_ref[0, 0] = lab


def _scores_labels(cls):
    s3, l3 = pl.pallas_call(
        _score_label_body,
        grid=(_B,),
        in_specs=[pl.BlockSpec((1, _N, _C), lambda i: (i, 0, 0))],
        out_specs=[pl.BlockSpec((1, 1, _N), lambda i: (i, 0, 0)),
                   pl.BlockSpec((1, 1, _N), lambda i: (i, 0, 0))],
        out_shape=[jax.ShapeDtypeStruct((_B, 1, _N), jnp.float32),
                   jax.ShapeDtypeStruct((_B, 1, _N), jnp.float32)],
    )(cls)
    return s3[:, 0, :], l3[:, 0, :]


def _nms_body(x1h, y1h, x2h, y2h, sh, lh, oh,
              ox1h, oy1h, ox2h, oy2h, osh, olh,
              ux1, uy1, ux2, uy2, us, ul, ordv,
              sx1, sy1, sx2, sy2, ss, slb,
              ox1, oy1, ox2, oy2, osv, olv):
    cid = lax.axis_index("c")
    sid = lax.axis_index("s")
    wid = sid * 2 + cid
    img = wid // 4
    sub = wid % 4
    is_leader = sub == 0
    my_lo = _RB + sub * _STRIPE
    my_hi = my_lo + _STRIPE

    pltpu.sync_copy(x1h.at[img], ux1)
    pltpu.sync_copy(y1h.at[img], uy1)
    pltpu.sync_copy(x2h.at[img], ux2)
    pltpu.sync_copy(y2h.at[img], uy2)
    pltpu.sync_copy(sh.at[img], us)
    pltpu.sync_copy(lh.at[img], ul)
    pltpu.sync_copy(oh.at[img], ordv)

    iota = lax.broadcasted_iota(jnp.int32, (16,), 0)
    ninf = jnp.full((16,), -jnp.inf, jnp.float32)

    def init(j, nvalid):
        sl16 = pl.ds(j * 16, 16)
        idx = ordv[sl16]
        sx1[sl16] = plsc.load_gather(ux1, [idx])
        sy1[sl16] = plsc.load_gather(uy1, [idx])
        sx2[sl16] = plsc.load_gather(ux2, [idx])
        sy2[sl16] = plsc.load_gather(uy2, [idx])
        sv = plsc.load_gather(us, [idx])
        ss[sl16] = sv
        slb[sl16] = plsc.load_gather(ul, [idx])
        return nvalid + jnp.sum((sv > _NEG_INF).astype(jnp.int32))

    n_valid = lax.fori_loop(0, _NCH, init, jnp.int32(0))

    @pl.when(is_leader)
    def _():
        @pl.loop(0, _OUTP // 16)
        def _(j):
            sl16 = pl.ds(j * 16, 16)
            neg = jnp.full((16,), -1.0, jnp.float32)
            ox1[sl16] = neg
            oy1[sl16] = neg
            ox2[sl16] = neg
            oy2[sl16] = neg
            osv[sl16] = neg
            olv[sl16] = neg

    def extract(pos):
        base = (pos // 16) * 16
        lsel = iota == (pos - base)
        sl16 = pl.ds(base, 16)
        sp = jnp.max(jnp.where(lsel, ss[sl16], -jnp.inf))
        x1p = jnp.max(jnp.where(lsel, sx1[sl16], -jnp.inf))
        y1p = jnp.max(jnp.where(lsel, sy1[sl16], -jnp.inf))
        x2p = jnp.max(jnp.where(lsel, sx2[sl16], -jnp.inf))
        y2p = jnp.max(jnp.where(lsel, sy2[sl16], -jnp.inf))
        return sp, (x1p, y1p, x2p, y2p, (x2p - x1p) * (y2p - y1p))

    def blockw(box_p, jbase, masked, pos):
        x1p, y1p, x2p, y2p, ap = box_p
        nsub = _BLK // 16
        xs = []
        for k in range(nsub):
            slj = pl.ds(jbase + k * 16, 16)
            xs.append((sx1[slj], sy1[slj], sx2[slj], sy2[slj]))
        supps = []
        for k in range(nsub):
            x1c, y1c, x2c, y2c = xs[k]
            xx1 = jnp.maximum(x1p, x1c)
            yy1 = jnp.maximum(y1p, y1c)
            xx2 = jnp.minimum(x2p, x2c)
            yy2 = jnp.minimum(y2p, y2c)
            inter = (jnp.maximum(xx2 - xx1, 0.0)
                     * jnp.maximum(yy2 - yy1, 0.0))
            union = (ap + (x2c - x1c) * (y2c - y1c)) - inter
            supp = inter > _NMS_TH * jnp.maximum(union, 1e-12)
            if masked:
                supp = supp & ((jbase + k * 16 + iota) > pos)
            supps.append(supp)
        for k in range(nsub):
            idx = jbase + k * 16 + iota
            plsc.store_scatter(ss, [idx], ninf, mask=supps[k])

    def make_scan(ranges_fn):
        def scan_body(carry):
            pos, cnt = carry
            sp, box_p = extract(pos)
            is_kept = sp > _NEG_INF

            @pl.when(is_kept & (cnt < _MAXD - 1))
            def _():
                blockw(box_p, (pos // _BLK) * _BLK, True, pos)

                def blk(j, c):
                    blockw(box_p, j * _BLK, False, pos)
                    return c

                for lo, hi in ranges_fn(pos):
                    lax.fori_loop(lo, hi, blk, jnp.int32(0))

            cnt2 = cnt + jnp.where(is_kept, 1, 0).astype(jnp.int32)
            return (pos + 1, cnt2)
        return scan_body

    scan_a = make_scan(lambda pos: [(pos // _BLK + 1, _RB),
                                    (my_lo, my_hi)])
    lim_a = jnp.minimum(n_valid, _R)
    pos_a, cnt_a = lax.while_loop(
        lambda c: (c[0] < lim_a) & (c[1] < _MAXD),
        scan_a, (jnp.int32(0), jnp.int32(0)))

    @pl.when(is_leader & (cnt_a < _MAXD) & (n_valid > _R))
    def _():
        def resup(p2, c):
            sp, box_p = extract(p2)

            @pl.when(sp > _NEG_INF)
            def _():
                def blk(j, cc):
                    blockw(box_p, j * _BLK, False, p2)
                    return cc
                lax.fori_loop(_RB, _NB, blk, jnp.int32(0))
            return c

        lax.fori_loop(0, _R, resup, jnp.int32(0))

        scan_b = make_scan(lambda pos: [(pos // _BLK + 1, _NB)])
        lax.while_loop(
            lambda c: (c[0] < n_valid) & (c[1] < _MAXD),
            scan_b, (jnp.int32(_R), cnt_a))

    @pl.when(is_leader)
    def _():
        def comp(j, off):
            slj = pl.ds(j * 16, 16)
            m = (ss[slj] > _NEG_INF).astype(jnp.int32)
            csum = plsc.cumsum(m)
            dest = off + csum - 1
            sel = (m > 0) & (dest < _MAXD)
            plsc.store_scatter(ox1, [dest], sx1[slj], mask=sel)
            plsc.store_scatter(oy1, [dest], sy1[slj], mask=sel)
            plsc.store_scatter(ox2, [dest], sx2[slj], mask=sel)
            plsc.store_scatter(oy2, [dest], sy2[slj], mask=sel)
            plsc.store_scatter(osv, [dest], ss[slj], mask=sel)
            plsc.store_scatter(olv, [dest], slb[slj], mask=sel)
            return off + jnp.max(csum)

        lax.fori_loop(0, _NCH, comp, jnp.int32(0))

        pltpu.sync_copy(ox1, ox1h.at[img])
        pltpu.sync_copy(oy1, oy1h.at[img])
        pltpu.sync_copy(ox2, ox2h.at[img])
        pltpu.sync_copy(oy2, oy2h.at[img])
        pltpu.sync_copy(osv, osh.at[img])
        pltpu.sync_copy(olv, olh.at[img])


_sc_params = pltpu.CompilerParams()
if "needs_layout_passes" in pltpu.CompilerParams.__dataclass_fields__:
    _sc_params = dataclasses.replace(_sc_params, needs_layout_passes=False)

_nms_call = functools.partial(
    pl.kernel,
    compiler_params=_sc_params,
    out_type=[jax.ShapeDtypeStruct((_B, _OUTP), jnp.float32)
              for _ in range(6)],
    mesh=plsc.VectorSubcoreMesh(core_axis_name="c", subcore_axis_name="s"),
    scratch_types=(
        [pltpu.VMEM((_NP,), jnp.float32) for _ in range(6)]
        + [pltpu.VMEM((_NP,), jnp.int32)]
        + [pltpu.VMEM((_NP,), jnp.float32) for _ in range(6)]
        + [pltpu.VMEM((_OUTP,), jnp.float32) for _ in range(6)]
    ),
)(_nms_body)


def kernel(box, cls):
    s, lab = _scores_labels(cls)
    order = jnp.argsort(-s, axis=1).astype(jnp.int32)
    pad_idx = jnp.broadcast_to(jnp.arange(_N, _NP, dtype=jnp.int32),
                               (_B, _PAD))
    order_p = jnp.concatenate([order, pad_idx], axis=1)
    s_p = jnp.concatenate(
        [s, jnp.full((_B, _PAD), -jnp.inf, jnp.float32)], axis=1)
    lab_p = jnp.concatenate([lab, jnp.zeros((_B, _PAD), jnp.float32)], axis=1)
    box_p = jnp.concatenate(
        [box, jnp.zeros((_B, _PAD, 4), jnp.float32)], axis=1)
    x1 = box_p[:, :, 0]
    y1 = box_p[:, :, 1]
    x2 = box_p[:, :, 2]
    y2 = box_p[:, :, 3]
    ox1, oy1, ox2, oy2, osv, olv = _nms_call(
        x1, y1, x2, y2, s_p, lab_p, order_p)
    boxes = jnp.stack([ox1[:, :_MAXD], oy1[:, :_MAXD],
                       ox2[:, :_MAXD], oy2[:, :_MAXD]], axis=-1)
    scores = osv[:, :_MAXD, None]
    labels = olv[:, :_MAXD, None]
    return boxes, scores, labels


# speedup vs baseline: 510.4681x; 1.2032x over previous
import dataclasses
import functools

import jax
import jax.numpy as jnp
from jax import lax
from jax.experimental import pallas as pl
from jax.experimental.pallas import tpu as pltpu
from jax.experimental.pallas import tpu_sc as plsc

_N = 5000
_C = 80
_B = 8
_PAD = 120
_NP = _N + _PAD
_NCH = _NP // 16
_BLK = 128
_NB = _NP // _BLK
_R = 512
_RB = _R // _BLK
_STRIPE = (_NB - _RB) // 4
_MAXD = 300
_OUTP = 304
_NMS_TH = 0.5
_SCORE_TH = 0.05
_NEG_INF = float("-inf")


def _score_label_body(cls_ref, s_ref, l_ref):
    c = cls_ref[0]
    m = jnp.max(c, axis=0)
    cls_ids = lax.broadcasted_iota(jnp.int32, (_C, _N), 0)
    lab = jnp.min(jnp.where(c == m[None, :], cls_ids, _C),
                  axis=0).astype(jnp.float32)
    s_ref[0, 0] = jnp.where(m > _SCORE_TH, m, -jnp.inf)
    l_ref[0, 0] = lab


def _scores_labels(cls):
    cls_t = cls.transpose(0, 2, 1)
    s3, l3 = pl.pallas_call(
        _score_label_body,
        grid=(_B,),
        in_specs=[pl.BlockSpec((1, _C, _N), lambda i: (i, 0, 0))],
        out_specs=[pl.BlockSpec((1, 1, _N), lambda i: (i, 0, 0)),
                   pl.BlockSpec((1, 1, _N), lambda i: (i, 0, 0))],
        out_shape=[jax.ShapeDtypeStruct((_B, 1, _N), jnp.float32),
                   jax.ShapeDtypeStruct((_B, 1, _N), jnp.float32)],
    )(cls_t)
    return s3[:, 0, :], l3[:, 0, :]


def _nms_body(x1h, y1h, x2h, y2h, sh, lh, oh,
              ox1h, oy1h, ox2h, oy2h, osh, olh,
              ux1, uy1, ux2, uy2, us, ul, ordv,
              sx1, sy1, sx2, sy2, ss, slb,
              ox1, oy1, ox2, oy2, osv, olv):
    cid = lax.axis_index("c")
    sid = lax.axis_index("s")
    wid = sid * 2 + cid
    img = wid // 4
    sub = wid % 4
    is_leader = sub == 0
    my_lo = _RB + sub * _STRIPE
    my_hi = my_lo + _STRIPE

    pltpu.sync_copy(x1h.at[img], ux1)
    pltpu.sync_copy(y1h.at[img], uy1)
    pltpu.sync_copy(x2h.at[img], ux2)
    pltpu.sync_copy(y2h.at[img], uy2)
    pltpu.sync_copy(sh.at[img], us)
    pltpu.sync_copy(lh.at[img], ul)
    pltpu.sync_copy(oh.at[img], ordv)

    iota = lax.broadcasted_iota(jnp.int32, (16,), 0)
    ninf = jnp.full((16,), -jnp.inf, jnp.float32)

    def init(j, nvalid):
        sl16 = pl.ds(j * 16, 16)
        idx = ordv[sl16]
        sx1[sl16] = plsc.load_gather(ux1, [idx])
        sy1[sl16] = plsc.load_gather(uy1, [idx])
        sx2[sl16] = plsc.load_gather(ux2, [idx])
        sy2[sl16] = plsc.load_gather(uy2, [idx])
        sv = plsc.load_gather(us, [idx])
        ss[sl16] = sv
        slb[sl16] = plsc.load_gather(ul, [idx])
        return nvalid + jnp.sum((sv > _NEG_INF).astype(jnp.int32))

    n_valid = lax.fori_loop(0, _NCH, init, jnp.int32(0))

    @pl.when(is_leader)
    def _():
        @pl.loop(0, _OUTP // 16)
        def _(j):
            sl16 = pl.ds(j * 16, 16)
            neg = jnp.full((16,), -1.0, jnp.float32)
            ox1[sl16] = neg
            oy1[sl16] = neg
            ox2[sl16] = neg
            oy2[sl16] = neg
            osv[sl16] = neg
            olv[sl16] = neg

    def extract(pos):
        base = (pos // 16) * 16
        lsel = iota == (pos - base)
        sl16 = pl.ds(base, 16)
        sp = jnp.max(jnp.where(lsel, ss[sl16], -jnp.inf))
        x1p = jnp.max(jnp.where(lsel, sx1[sl16], -jnp.inf))
        y1p = jnp.max(jnp.where(lsel, sy1[sl16], -jnp.inf))
        x2p = jnp.max(jnp.where(lsel, sx2[sl16], -jnp.inf))
        y2p = jnp.max(jnp.where(lsel, sy2[sl16], -jnp.inf))
        return sp, (x1p, y1p, x2p, y2p, (x2p - x1p) * (y2p - y1p))

    def blockw(box_p, jbase, masked, pos):
        x1p, y1p, x2p, y2p, ap = box_p
        nsub = _BLK // 16
        xs = []
        for k in range(nsub):
            slj = pl.ds(jbase + k * 16, 16)
            xs.append((sx1[slj], sy1[slj], sx2[slj], sy2[slj]))
        supps = []
        for k in range(nsub):
            x1c, y1c, x2c, y2c = xs[k]
            xx1 = jnp.maximum(x1p, x1c)
            yy1 = jnp.maximum(y1p, y1c)
            xx2 = jnp.minimum(x2p, x2c)
            yy2 = jnp.minimum(y2p, y2c)
            inter = (jnp.maximum(xx2 - xx1, 0.0)
                     * jnp.maximum(yy2 - yy1, 0.0))
            union = (ap + (x2c - x1c) * (y2c - y1c)) - inter
            supp = inter > _NMS_TH * jnp.maximum(union, 1e-12)
            if masked:
                supp = supp & ((jbase + k * 16 + iota) > pos)
            supps.append(supp)
        for k in range(nsub):
            idx = jbase + k * 16 + iota
            plsc.store_scatter(ss, [idx], ninf, mask=supps[k])

    def make_scan(ranges_fn):
        def scan_body(carry):
            pos, cnt = carry
            sp, box_p = extract(pos)
            is_kept = sp > _NEG_INF

            @pl.when(is_kept & (cnt < _MAXD - 1))
            def _():
                blockw(box_p, (pos // _BLK) * _BLK, True, pos)

                def blk(j, c):
                    blockw(box_p, j * _BLK, False, pos)
                    return c

                for lo, hi in ranges_fn(pos):
                    lax.fori_loop(lo, hi, blk, jnp.int32(0))

            cnt2 = cnt + jnp.where(is_kept, 1, 0).astype(jnp.int32)
            return (pos + 1, cnt2)
        return scan_body

    scan_a = make_scan(lambda pos: [(pos // _BLK + 1, _RB),
                                    (my_lo, my_hi)])
    lim_a = jnp.minimum(n_valid, _R)
    pos_a, cnt_a = lax.while_loop(
        lambda c: (c[0] < lim_a) & (c[1] < _MAXD),
        scan_a, (jnp.int32(0), jnp.int32(0)))

    @pl.when(is_leader & (cnt_a < _MAXD) & (n_valid > _R))
    def _():
        def resup(p2, c):
            sp, box_p = extract(p2)

            @pl.when(sp > _NEG_INF)
            def _():
                def blk(j, cc):
                    blockw(box_p, j * _BLK, False, p2)
                    return cc
                lax.fori_loop(_RB, _NB, blk, jnp.int32(0))
            return c

        lax.fori_loop(0, _R, resup, jnp.int32(0))

        scan_b = make_scan(lambda pos: [(pos // _BLK + 1, _NB)])
        lax.while_loop(
            lambda c: (c[0] < n_valid) & (c[1] < _MAXD),
            scan_b, (jnp.int32(_R), cnt_a))

    @pl.when(is_leader)
    def _():
        def comp(j, off):
            slj = pl.ds(j * 16, 16)
            m = (ss[slj] > _NEG_INF).astype(jnp.int32)
            csum = plsc.cumsum(m)
            dest = off + csum - 1
            sel = (m > 0) & (dest < _MAXD)
            plsc.store_scatter(ox1, [dest], sx1[slj], mask=sel)
            plsc.store_scatter(oy1, [dest], sy1[slj], mask=sel)
            plsc.store_scatter(ox2, [dest], sx2[slj], mask=sel)
            plsc.store_scatter(oy2, [dest], sy2[slj], mask=sel)
            plsc.store_scatter(osv, [dest], ss[slj], mask=sel)
            plsc.store_scatter(olv, [dest], slb[slj], mask=sel)
            return off + jnp.max(csum)

        lax.fori_loop(0, _NCH, comp, jnp.int32(0))

        pltpu.sync_copy(ox1, ox1h.at[img])
        pltpu.sync_copy(oy1, oy1h.at[img])
        pltpu.sync_copy(ox2, ox2h.at[img])
        pltpu.sync_copy(oy2, oy2h.at[img])
        pltpu.sync_copy(osv, osh.at[img])
        pltpu.sync_copy(olv, olh.at[img])


_sc_params = pltpu.CompilerParams()
if "needs_layout_passes" in pltpu.CompilerParams.__dataclass_fields__:
    _sc_params = dataclasses.replace(_sc_params, needs_layout_passes=False)

_nms_call = functools.partial(
    pl.kernel,
    compiler_params=_sc_params,
    out_type=[jax.ShapeDtypeStruct((_B, _OUTP), jnp.float32)
              for _ in range(6)],
    mesh=plsc.VectorSubcoreMesh(core_axis_name="c", subcore_axis_name="s"),
    scratch_types=(
        [pltpu.VMEM((_NP,), jnp.float32) for _ in range(6)]
        + [pltpu.VMEM((_NP,), jnp.int32)]
        + [pltpu.VMEM((_NP,), jnp.float32) for _ in range(6)]
        + [pltpu.VMEM((_OUTP,), jnp.float32) for _ in range(6)]
    ),
)(_nms_body)


def kernel(box, cls):
    s, lab = _scores_labels(cls)
    order = jnp.argsort(-s, axis=1).astype(jnp.int32)
    pad_idx = jnp.broadcast_to(jnp.arange(_N, _NP, dtype=jnp.int32),
                               (_B, _PAD))
    order_p = jnp.concatenate([order, pad_idx], axis=1)
    s_p = jnp.concatenate(
        [s, jnp.full((_B, _PAD), -jnp.inf, jnp.float32)], axis=1)
    lab_p = jnp.concatenate([lab, jnp.zeros((_B, _PAD), jnp.float32)], axis=1)
    box_p = jnp.concatenate(
        [box, jnp.zeros((_B, _PAD, 4), jnp.float32)], axis=1)
    x1 = box_p[:, :, 0]
    y1 = box_p[:, :, 1]
    x2 = box_p[:, :, 2]
    y2 = box_p[:, :, 3]
    ox1, oy1, ox2, oy2, osv, olv = _nms_call(
        x1, y1, x2, y2, s_p, lab_p, order_p)
    boxes = jnp.stack([ox1[:, :_MAXD], oy1[:, :_MAXD],
                       ox2[:, :_MAXD], oy2[:, :_MAXD]], axis=-1)
    scores = osv[:, :_MAXD, None]
    labels = olv[:, :_MAXD, None]
    return boxes, scores, labels
